# Initial kernel scaffold; baseline (speedup 1.0000x reference)
import jax
import jax.numpy as jnp
from jax.experimental import pallas as pl


def kernel(pred, label):
    raise NotImplementedError("write your pallas kernel here")


# speedup vs baseline: 7.8501x; 7.8501x over previous
import jax
import jax.numpy as jnp
from jax.experimental import pallas as pl
from jax.experimental.pallas import tpu as pltpu

_V = 100000
_C = 128
_BR = 2000
_NB = _V // _BR


def _ce_body(pred_ref, label_ref, out_ref, cnt_acc, s_acc):
    i = pl.program_id(0)

    @pl.when(i == 0)
    def _init():
        cnt_acc[...] = jnp.zeros_like(cnt_acc)
        s_acc[...] = jnp.zeros_like(s_acc)

    x = pred_ref[...]
    m = jnp.max(x, axis=1, keepdims=True)
    s = jnp.sum(jnp.exp(x - m), axis=1, keepdims=True)
    lse = m + jnp.log(s)
    lab = label_ref[0, 0, :]
    col = jax.lax.broadcasted_iota(jnp.int32, (_BR, _C), 1)
    oh = (col == lab[:, None]).astype(jnp.float32)
    p = jnp.sum(x * oh, axis=1, keepdims=True)
    nll = lse - p
    cnt_acc[...] += jnp.sum(oh, axis=0, keepdims=True)
    s_acc[...] += jnp.sum(oh * nll, axis=0, keepdims=True)

    @pl.when(i == _NB - 1)
    def _fin():
        cs = cnt_acc[...]
        w = (_V - cs) * (1.0 / _V) * (cs > 0).astype(jnp.float32)
        num = jnp.sum(w * s_acc[...])
        den = jnp.sum(w * cs)
        out_ref[...] = jnp.reshape(num / den, (1, 1))


def kernel(pred, label):
    lab3 = label.astype(jnp.int32).reshape(_NB, 1, _BR)
    out = pl.pallas_call(
        _ce_body,
        grid=(_NB,),
        in_specs=[
            pl.BlockSpec((_BR, _C), lambda i: (i, 0)),
            pl.BlockSpec((1, 1, _BR), lambda i: (i, 0, 0)),
        ],
        out_specs=pl.BlockSpec((1, 1), lambda i: (0, 0)),
        out_shape=jax.ShapeDtypeStruct((1, 1), jnp.float32),
        scratch_shapes=[
            pltpu.VMEM((1, _C), jnp.float32),
            pltpu.VMEM((1, _C), jnp.float32),
        ],
        compiler_params=pltpu.CompilerParams(
            dimension_semantics=("arbitrary",)
        ),
    )(pred, lab3)
    return out[0, 0]


# speedup vs baseline: 10.5896x; 1.3490x over previous
import jax
import jax.numpy as jnp
from jax.experimental import pallas as pl
from jax.experimental.pallas import tpu as pltpu

_V = 100000
_C = 128
_BR = 2000
_NB = _V // _BR


def _ce_body(pred_ref, label_ref, out_ref, cnt_acc, s_acc):
    i = pl.program_id(0)

    @pl.when(i == 0)
    def _init():
        cnt_acc[...] = jnp.zeros_like(cnt_acc)
        s_acc[...] = jnp.zeros_like(s_acc)

    x = pred_ref[...]
    m = jnp.max(x, axis=1, keepdims=True)
    d = x - m
    e = jnp.exp(d)
    ones_col = jnp.ones((_C, 1), jnp.float32)
    s = jax.lax.dot_general(e, ones_col, (((1,), (0,)), ((), ())),
                            preferred_element_type=jnp.float32)
    logs = jnp.log(s)
    lab = label_ref[0, 0, :]
    col = jax.lax.broadcasted_iota(jnp.int32, (_BR, _C), 1)
    is_lab = col == lab[:, None]
    oh = is_lab.astype(jnp.float32)
    z = jnp.where(is_lab, d, 0.0)
    ones_row = jnp.ones((1, _BR), jnp.float32)
    cnt_part = jax.lax.dot_general(ones_row, oh, (((1,), (0,)), ((), ())),
                                   preferred_element_type=jnp.float32)
    l_part = jax.lax.dot_general(logs, oh, (((0,), (0,)), ((), ())),
                                 preferred_element_type=jnp.float32)
    d_part = jax.lax.dot_general(ones_row, z, (((1,), (0,)), ((), ())),
                                 preferred_element_type=jnp.float32)
    cnt_acc[...] += cnt_part
    s_acc[...] += l_part - d_part

    @pl.when(i == _NB - 1)
    def _fin():
        cs = cnt_acc[...]
        w = (_V - cs) * (1.0 / _V) * (cs > 0).astype(jnp.float32)
        num = jnp.sum(w * s_acc[...])
        den = jnp.sum(w * cs)
        out_ref[...] = jnp.reshape(num / den, (1, 1))


def kernel(pred, label):
    lab3 = label.astype(jnp.int32).reshape(_NB, 1, _BR)
    out = pl.pallas_call(
        _ce_body,
        grid=(_NB,),
        in_specs=[
            pl.BlockSpec((_BR, _C), lambda i: (i, 0)),
            pl.BlockSpec((1, 1, _BR), lambda i: (i, 0, 0)),
        ],
        out_specs=pl.BlockSpec((1, 1), lambda i: (0, 0)),
        out_shape=jax.ShapeDtypeStruct((1, 1), jnp.float32),
        scratch_shapes=[
            pltpu.VMEM((1, _C), jnp.float32),
            pltpu.VMEM((1, _C), jnp.float32),
        ],
        compiler_params=pltpu.CompilerParams(
            dimension_semantics=("arbitrary",)
        ),
    )(pred, lab3)
    return out[0, 0]


# speedup vs baseline: 12.5780x; 1.1878x over previous
import jax
import jax.numpy as jnp
from jax.experimental import pallas as pl
from jax.experimental.pallas import tpu as pltpu

_V = 100000
_C = 128
_BR = 5000
_NB = _V // _BR


def _ce_body(pred_ref, label_ref, out_ref, cnt_acc, s_acc):
    i = pl.program_id(0)

    @pl.when(i == 0)
    def _init():
        cnt_acc[...] = jnp.zeros_like(cnt_acc)
        s_acc[...] = jnp.zeros_like(s_acc)

    x = pred_ref[...]
    m = jnp.max(x, axis=1, keepdims=True)
    d = x - m
    e = jnp.exp(d)
    ones_col = jnp.ones((_C, 1), jnp.float32)
    s = jax.lax.dot_general(e, ones_col, (((1,), (0,)), ((), ())),
                            preferred_element_type=jnp.float32)
    logs = jnp.log(s)
    lab = label_ref[0, 0, :]
    col = jax.lax.broadcasted_iota(jnp.int32, (_BR, _C), 1)
    is_lab = col == lab[:, None]
    oh = is_lab.astype(jnp.float32)
    z = jnp.where(is_lab, d, 0.0)
    ones_row = jnp.ones((1, _BR), jnp.float32)
    cnt_part = jax.lax.dot_general(ones_row, oh, (((1,), (0,)), ((), ())),
                                   preferred_element_type=jnp.float32)
    l_part = jax.lax.dot_general(logs, oh, (((0,), (0,)), ((), ())),
                                 preferred_element_type=jnp.float32)
    d_part = jax.lax.dot_general(ones_row, z, (((1,), (0,)), ((), ())),
                                 preferred_element_type=jnp.float32)
    cnt_acc[...] += cnt_part
    s_acc[...] += l_part - d_part

    @pl.when(i == _NB - 1)
    def _fin():
        cs = cnt_acc[...]
        w = (_V - cs) * (1.0 / _V) * (cs > 0).astype(jnp.float32)
        num = jnp.sum(w * s_acc[...])
        den = jnp.sum(w * cs)
        out_ref[...] = jnp.reshape(num / den, (1, 1))


def kernel(pred, label):
    lab3 = label.astype(jnp.int32).reshape(_NB, 1, _BR)
    out = pl.pallas_call(
        _ce_body,
        grid=(_NB,),
        in_specs=[
            pl.BlockSpec((_BR, _C), lambda i: (i, 0)),
            pl.BlockSpec((1, 1, _BR), lambda i: (i, 0, 0)),
        ],
        out_specs=pl.BlockSpec((1, 1), lambda i: (0, 0)),
        out_shape=jax.ShapeDtypeStruct((1, 1), jnp.float32),
        scratch_shapes=[
            pltpu.VMEM((1, _C), jnp.float32),
            pltpu.VMEM((1, _C), jnp.float32),
        ],
        compiler_params=pltpu.CompilerParams(
            dimension_semantics=("arbitrary",)
        ),
    )(pred, lab3)
    return out[0, 0]


# speedup vs baseline: 14.1497x; 1.1250x over previous
import jax
import jax.numpy as jnp
from jax.experimental import pallas as pl
from jax.experimental.pallas import tpu as pltpu

_V = 100000
_C = 128
_BR = 10000
_NB = _V // _BR


def _ce_body(pred_ref, label_ref, out_ref, cnt_acc, s_acc):
    i = pl.program_id(0)

    @pl.when(i == 0)
    def _init():
        cnt_acc[...] = jnp.zeros_like(cnt_acc)
        s_acc[...] = jnp.zeros_like(s_acc)

    x = pred_ref[...]
    m = jnp.max(x, axis=1, keepdims=True)
    d = x - m
    e = jnp.exp(d)
    ones_col = jnp.ones((_C, 1), jnp.float32)
    s = jax.lax.dot_general(e, ones_col, (((1,), (0,)), ((), ())),
                            preferred_element_type=jnp.float32)
    logs = jnp.log(s)
    lab = label_ref[0, 0, :]
    col = jax.lax.broadcasted_iota(jnp.int32, (_BR, _C), 1)
    is_lab = col == lab[:, None]
    oh = is_lab.astype(jnp.float32)
    z = jnp.where(is_lab, d, 0.0)
    ones_row = jnp.ones((1, _BR), jnp.float32)
    cnt_part = jax.lax.dot_general(ones_row, oh, (((1,), (0,)), ((), ())),
                                   preferred_element_type=jnp.float32)
    l_part = jax.lax.dot_general(logs, oh, (((0,), (0,)), ((), ())),
                                 preferred_element_type=jnp.float32)
    d_part = jax.lax.dot_general(ones_row, z, (((1,), (0,)), ((), ())),
                                 preferred_element_type=jnp.float32)
    cnt_acc[...] += cnt_part
    s_acc[...] += l_part - d_part

    @pl.when(i == _NB - 1)
    def _fin():
        cs = cnt_acc[...]
        w = (_V - cs) * (1.0 / _V) * (cs > 0).astype(jnp.float32)
        num = jnp.sum(w * s_acc[...])
        den = jnp.sum(w * cs)
        out_ref[...] = jnp.reshape(num / den, (1, 1))


def kernel(pred, label):
    lab3 = label.astype(jnp.int32).reshape(_NB, 1, _BR)
    out = pl.pallas_call(
        _ce_body,
        grid=(_NB,),
        in_specs=[
            pl.BlockSpec((_BR, _C), lambda i: (i, 0)),
            pl.BlockSpec((1, 1, _BR), lambda i: (i, 0, 0)),
        ],
        out_specs=pl.BlockSpec((1, 1), lambda i: (0, 0)),
        out_shape=jax.ShapeDtypeStruct((1, 1), jnp.float32),
        scratch_shapes=[
            pltpu.VMEM((1, _C), jnp.float32),
            pltpu.VMEM((1, _C), jnp.float32),
        ],
        compiler_params=pltpu.CompilerParams(
            dimension_semantics=("arbitrary",)
        ),
    )(pred, lab3)
    return out[0, 0]
